# initial kernel scaffold (unmeasured)
import jax
import jax.numpy as jnp
from jax import lax
from jax.experimental import pallas as pl
from jax.experimental.pallas import tpu as pltpu

M = 8192
D = 2048
BLK = 512
NBLK = M // BLK


def kernel(partial, resid, gamma):
    gamma2 = gamma.reshape(1, D)

    def body(
        x_ref,
        resid_ref,
        gamma_ref,
        out_ref,
        acc_ref,
        recv1_ref,
        recv2_ref,
        va, vb, vc, vo,
        copy_sems,
        send_sem1, recv_sem1, send_sem2, recv_sem2,
    ):
        my_x = lax.axis_index("x")
        my_y = lax.axis_index("y")
        my_z = lax.axis_index("z")

        rdma1 = pltpu.make_async_remote_copy(
            src_ref=x_ref.at[0],
            dst_ref=recv1_ref,
            send_sem=send_sem1,
            recv_sem=recv_sem1,
            device_id=(my_x, my_y, my_z ^ 1),
            device_id_type=pl.DeviceIdType.MESH,
        )
        rdma1.start()
        rdma1.wait()

        def sum_blk(i, carry):
            rows = pl.ds(i * BLK, BLK)
            cp1 = pltpu.make_async_copy(x_ref.at[0, rows, :], va, copy_sems.at[0])
            cp2 = pltpu.make_async_copy(recv1_ref.at[rows, :], vb, copy_sems.at[1])
            cp1.start()
            cp2.start()
            cp1.wait()
            cp2.wait()
            vo[...] = va[...] + vb[...]
            cp3 = pltpu.make_async_copy(vo, acc_ref.at[rows, :], copy_sems.at[2])
            cp3.start()
            cp3.wait()
            return carry

        lax.fori_loop(0, NBLK, sum_blk, 0)

        rdma2 = pltpu.make_async_remote_copy(
            src_ref=acc_ref,
            dst_ref=recv2_ref,
            send_sem=send_sem2,
            recv_sem=recv_sem2,
            device_id=(my_x, my_y, my_z ^ 2),
            device_id_type=pl.DeviceIdType.MESH,
        )
        rdma2.start()
        rdma2.wait()

        def fin_blk(i, carry):
            rows = pl.ds(i * BLK, BLK)
            cp1 = pltpu.make_async_copy(acc_ref.at[rows, :], va, copy_sems.at[0])
            cp2 = pltpu.make_async_copy(recv2_ref.at[rows, :], vb, copy_sems.at[1])
            cp3 = pltpu.make_async_copy(resid_ref.at[rows, :], vc, copy_sems.at[2])
            cp1.start()
            cp2.start()
            cp3.start()
            cp1.wait()
            cp2.wait()
            cp3.wait()
            y = va[...] + vb[...] + vc[...]
            rms = jnp.sqrt(jnp.mean(y * y, axis=-1, keepdims=True) + 1e-6)
            vo[...] = y / rms * gamma_ref[...]
            cp4 = pltpu.make_async_copy(vo, out_ref.at[rows, :], copy_sems.at[3])
            cp4.start()
            cp4.wait()
            return carry

        lax.fori_loop(0, NBLK, fin_blk, 0)

    out, _, _, _ = pl.pallas_call(
        body,
        out_shape=[
            jax.ShapeDtypeStruct((M, D), jnp.float32),
            jax.ShapeDtypeStruct((M, D), jnp.float32),
            jax.ShapeDtypeStruct((M, D), jnp.float32),
            jax.ShapeDtypeStruct((M, D), jnp.float32),
        ],
        in_specs=[
            pl.BlockSpec(memory_space=pltpu.ANY),
            pl.BlockSpec(memory_space=pltpu.ANY),
            pl.BlockSpec(memory_space=pltpu.VMEM),
        ],
        out_specs=[
            pl.BlockSpec(memory_space=pltpu.ANY),
            pl.BlockSpec(memory_space=pltpu.ANY),
            pl.BlockSpec(memory_space=pltpu.ANY),
            pl.BlockSpec(memory_space=pltpu.ANY),
        ],
        scratch_shapes=[
            pltpu.VMEM((BLK, D), jnp.float32),
            pltpu.VMEM((BLK, D), jnp.float32),
            pltpu.VMEM((BLK, D), jnp.float32),
            pltpu.VMEM((BLK, D), jnp.float32),
            pltpu.SemaphoreType.DMA((4,)),
            pltpu.SemaphoreType.DMA,
            pltpu.SemaphoreType.DMA,
            pltpu.SemaphoreType.DMA,
            pltpu.SemaphoreType.DMA,
        ],
        compiler_params=pltpu.CompilerParams(collective_id=0),
    )(partial, resid, gamma2)
    return out


# baseline (device time: 2429714 ns/iter reference)
import jax
import jax.numpy as jnp
from jax import lax
from jax.experimental import pallas as pl
from jax.experimental.pallas import tpu as pltpu

M = 8192
D = 2048
BLK = 512
NBLK = M // BLK


def kernel(partial, resid, gamma):
    gamma2 = gamma.reshape(1, D)

    def body(
        x_ref,
        resid_ref,
        gamma_ref,
        out_ref,
        acc_ref,
        recv1_ref,
        recv2_ref,
        va, vb, vc, vo,
        copy_sems,
        send_sem1, recv_sem1, send_sem2, recv_sem2,
    ):
        my_x = lax.axis_index("x")
        my_y = lax.axis_index("y")
        my_z = lax.axis_index("z")

        barrier_sem = pltpu.get_barrier_semaphore()
        for dz in (my_z ^ 1, my_z ^ 2):
            pl.semaphore_signal(
                barrier_sem,
                inc=1,
                device_id=(my_x, my_y, dz),
                device_id_type=pl.DeviceIdType.MESH,
            )
        pl.semaphore_wait(barrier_sem, 2)

        rdma1 = pltpu.make_async_remote_copy(
            src_ref=x_ref.at[0],
            dst_ref=recv1_ref,
            send_sem=send_sem1,
            recv_sem=recv_sem1,
            device_id=(my_x, my_y, my_z ^ 1),
            device_id_type=pl.DeviceIdType.MESH,
        )
        rdma1.start()
        rdma1.wait()

        def sum_blk(i, carry):
            rows = pl.ds(i * BLK, BLK)
            cp1 = pltpu.make_async_copy(x_ref.at[0, rows, :], va, copy_sems.at[0])
            cp2 = pltpu.make_async_copy(recv1_ref.at[rows, :], vb, copy_sems.at[1])
            cp1.start()
            cp2.start()
            cp1.wait()
            cp2.wait()
            vo[...] = va[...] + vb[...]
            cp3 = pltpu.make_async_copy(vo, acc_ref.at[rows, :], copy_sems.at[2])
            cp3.start()
            cp3.wait()
            return carry

        lax.fori_loop(0, NBLK, sum_blk, 0)

        rdma2 = pltpu.make_async_remote_copy(
            src_ref=acc_ref,
            dst_ref=recv2_ref,
            send_sem=send_sem2,
            recv_sem=recv_sem2,
            device_id=(my_x, my_y, my_z ^ 2),
            device_id_type=pl.DeviceIdType.MESH,
        )
        rdma2.start()
        rdma2.wait()

        def fin_blk(i, carry):
            rows = pl.ds(i * BLK, BLK)
            cp1 = pltpu.make_async_copy(acc_ref.at[rows, :], va, copy_sems.at[0])
            cp2 = pltpu.make_async_copy(recv2_ref.at[rows, :], vb, copy_sems.at[1])
            cp3 = pltpu.make_async_copy(resid_ref.at[rows, :], vc, copy_sems.at[2])
            cp1.start()
            cp2.start()
            cp3.start()
            cp1.wait()
            cp2.wait()
            cp3.wait()
            y = va[...] + vb[...] + vc[...]
            rms = jnp.sqrt(jnp.mean(y * y, axis=-1, keepdims=True) + 1e-6)
            vo[...] = y / rms * gamma_ref[...]
            cp4 = pltpu.make_async_copy(vo, out_ref.at[rows, :], copy_sems.at[3])
            cp4.start()
            cp4.wait()
            return carry

        lax.fori_loop(0, NBLK, fin_blk, 0)

    out, _, _, _ = pl.pallas_call(
        body,
        out_shape=[
            jax.ShapeDtypeStruct((M, D), jnp.float32),
            jax.ShapeDtypeStruct((M, D), jnp.float32),
            jax.ShapeDtypeStruct((M, D), jnp.float32),
            jax.ShapeDtypeStruct((M, D), jnp.float32),
        ],
        in_specs=[
            pl.BlockSpec(memory_space=pl.ANY),
            pl.BlockSpec(memory_space=pl.ANY),
            pl.BlockSpec(memory_space=pltpu.VMEM),
        ],
        out_specs=[
            pl.BlockSpec(memory_space=pl.ANY),
            pl.BlockSpec(memory_space=pl.ANY),
            pl.BlockSpec(memory_space=pl.ANY),
            pl.BlockSpec(memory_space=pl.ANY),
        ],
        scratch_shapes=[
            pltpu.VMEM((BLK, D), jnp.float32),
            pltpu.VMEM((BLK, D), jnp.float32),
            pltpu.VMEM((BLK, D), jnp.float32),
            pltpu.VMEM((BLK, D), jnp.float32),
            pltpu.SemaphoreType.DMA((4,)),
            pltpu.SemaphoreType.DMA,
            pltpu.SemaphoreType.DMA,
            pltpu.SemaphoreType.DMA,
            pltpu.SemaphoreType.DMA,
        ],
        compiler_params=pltpu.CompilerParams(collective_id=0),
    )(partial, resid, gamma2)
    return out
